# baseline (speedup 1.0000x reference)
import functools

import jax
import jax.numpy as jnp
from jax import lax
from jax.experimental import pallas as pl
from jax.experimental.pallas import tpu as pltpu
from jax.experimental.pallas import tpu_sc as plsc

f32 = jnp.float32
i32 = jnp.int32

N = 10000
NP = 10240
E2 = 170000
EP = 172032
P = 262144
C = 64
BR = 512
GR = NP // BR

_MESH = dict(core_axis_name="c", subcore_axis_name="s")


def _ln(y, g, b):
    mu = jnp.mean(y, axis=-1, keepdims=True)
    var = jnp.mean((y - mu) ** 2, axis=-1, keepdims=True)
    return (y - mu) / jnp.sqrt(var + 1e-5) * g + b


def _t1_body(x_ref, w_ref, b_ref, g_ref, be_ref, o_ref):
    xb = x_ref[...]
    xb = jnp.where(jnp.isfinite(xb), xb, 0.0)
    y = jnp.dot(xb, w_ref[...], preferred_element_type=f32) + b_ref[...]
    o_ref[...] = jnp.maximum(_ln(y, g_ref[...], be_ref[...]), 0.0)


def _t2_body(xp_ref, wl_ref, bl_ref, wr_ref, br_ref, wm1_ref, bm1_ref,
             lg_ref, lb_ref, wm2_ref, bm2_ref, m1_ref, cf_ref,
             xl_ref, xra_ref, zfc_ref, g16_ref):
    xp = xp_ref[...]
    xl = jnp.dot(xp, wl_ref[...], preferred_element_type=f32) + bl_ref[...]
    xr = jnp.dot(xp, wr_ref[...], preferred_element_type=f32) + br_ref[...]
    xl_ref[...] = xl
    hm = jnp.maximum(_ln(jnp.dot(xp, wm1_ref[...], preferred_element_type=f32)
                         + bm1_ref[...], lg_ref[...], lb_ref[...]), 0.0)
    zf = jnp.dot(hm, wm2_ref[...], preferred_element_type=f32) + bm2_ref[...]
    nf = jnp.maximum(jnp.sqrt(jnp.sum(zf * zf, axis=-1, keepdims=True)), 1e-8)
    zfc_ref[...] = zf / nf * cf_ref[0, 0]
    m1 = m1_ref[...]
    ar4 = jnp.dot(jnp.abs(xr), m1, preferred_element_type=f32)
    xra_ref[...] = jnp.concatenate(
        [xr, jnp.pad(ar4, ((0, 0), (0, 124)))], axis=1)
    al4 = jnp.dot(jnp.abs(xl), m1, preferred_element_type=f32)
    gb = jnp.pad(jnp.max(al4, axis=0, keepdims=True), ((0, 0), (0, 12)))

    @pl.when(pl.program_id(0) == 0)
    def _():
        g16_ref[...] = gb

    @pl.when(pl.program_id(0) > 0)
    def _():
        g16_ref[...] = jnp.maximum(g16_ref[...], gb)


def _t3_body(lo_ref, hi_ref, dl_ref, dh_ref, b1_ref, lg_ref, lb_ref,
             wl2_ref, bl2_ref, wr2_ref, br2_ref, a2_ref,
             xl2_ref, xra_ref, g16_ref):
    num = jnp.concatenate([lo_ref[...], hi_ref[...]], axis=1)
    dinv = 1.0 / (dl_ref[...][:, 0:4] + dh_ref[...][:, 0:4] + 1e-30)
    scale = jnp.concatenate(
        [jnp.broadcast_to(dinv[:, h:h + 1], (BR, 64)) for h in range(4)], axis=1)
    h1 = jnp.maximum(_ln(num * scale + b1_ref[...], lg_ref[...], lb_ref[...]), 0.0)
    xl2 = jnp.dot(h1, wl2_ref[...], preferred_element_type=f32) + bl2_ref[...]
    xr2 = jnp.dot(h1, wr2_ref[...], preferred_element_type=f32) + br2_ref[...]
    xl2_ref[...] = xl2
    a2 = a2_ref[...]
    ar1 = jnp.dot(jnp.abs(xr2), a2, preferred_element_type=f32)
    xra_ref[...] = jnp.concatenate(
        [xr2, jnp.pad(ar1, ((0, 0), (0, 127)))], axis=1)
    al1 = jnp.dot(jnp.abs(xl2), a2, preferred_element_type=f32)
    gb = jnp.pad(jnp.max(al1, axis=0, keepdims=True), ((0, 0), (0, 15)))

    @pl.when(pl.program_id(0) == 0)
    def _():
        g16_ref[...] = gb

    @pl.when(pl.program_id(0) > 0)
    def _():
        g16_ref[...] = jnp.maximum(g16_ref[...], gb)


def _t4_body(lo_ref, hi_ref, dl_ref, dh_ref, b2_ref, zfc_ref, cg_ref, t_ref,
             u_ref, ud_ref):
    num = lo_ref[...] + hi_ref[...]
    den = dl_ref[...][:, 0:1] + dh_ref[...][:, 0:1]
    zg = num / (den + 1e-30) + b2_ref[...]
    ng = jnp.maximum(jnp.sqrt(jnp.sum(zg * zg, axis=-1, keepdims=True)), 1e-8)
    u = jnp.concatenate([zg / ng * cg_ref[0, 0], zfc_ref[...]], axis=1)
    u_ref[...] = u
    ud_ref[...] = u * t_ref[0, 0]


def _full(shape):
    return pl.BlockSpec(shape, lambda i: tuple(0 for _ in shape))


def _rows(w):
    return pl.BlockSpec((BR, w), lambda i: (i, 0))


def _splat_i(v):
    return jnp.full((16,), v, i32)


_GDN = lax.GatherDimensionNumbers(offset_dims=(), collapsed_slice_dims=(0,),
                                  start_index_map=(0,))


def _lane_shuffle(v, idx):
    return lax.gather(v, idx[:, None], _GDN, (1,),
                      mode=lax.GatherScatterMode.PROMISE_IN_BOUNDS)


def _hsum16(v):
    iot = lax.iota(i32, 16)
    for sh in (8, 4, 2, 1):
        v = v + _lane_shuffle(v, iot ^ sh)
    return v


def _sc_pass1(xl_t, xra_t, g16, srcp, dstp, attf, heads, D, DA):
    HV = D // heads // 16
    PW = EP // 32
    NCH = PW // C
    ND8 = NP // 8
    ZR8 = ND8 // 16

    def body(xl_hbm, xra_hbm, g_hbm, src_hbm, dst_hbm, att_hbm, z_hbm,
             w_hbm, dlo_hbm, dhi_hbm,
             srcA, dstA, d8v, rl, rr, attv, gv, wbuf, val_d, acc_d,
             semA, semB):
        cid = lax.axis_index("c")
        sid = lax.axis_index("s")
        wid = sid * 2 + cid
        base0 = wid * PW
        pltpu.sync_copy(att_hbm, attv)
        pltpu.sync_copy(g_hbm, gv)
        pltpu.sync_copy(src_hbm.at[pl.ds(base0, PW)], srcA)
        pltpu.sync_copy(dst_hbm.at[pl.ds(base0, PW)], dstA)
        pltpu.sync_copy(z_hbm, acc_d.at[pl.ds(sid * ZR8, ZR8)])
        iot = lax.iota(i32, 16)
        plsc.subcore_barrier()

        gva = gv[pl.ds(0, 16)]
        ghs = [_lane_shuffle(gva, _splat_i(hh)) for hh in range(heads)]

        attr = [attv[pl.ds(v * 16, 16)] for v in range(D // 16)]
        sems = [semA, semB]

        def fetch(ci, b):
            off = ci * C
            pltpu.async_copy(xl_hbm.at[srcA.at[pl.ds(off, C)]], rl.at[b],
                             sems[b])
            pltpu.async_copy(xra_hbm.at[dstA.at[pl.ds(off, C)]], rr.at[b],
                             sems[b])

        fetch(0, 0)

        def pair(c2, _):
            for b in range(2):
                ci = c2 * 2 + b
                off = ci * C
                pltpu.make_async_copy(xl_hbm.at[srcA.at[pl.ds(off, C)]],
                                      rl.at[b], sems[b]).wait()
                pltpu.make_async_copy(xra_hbm.at[dstA.at[pl.ds(off, C)]],
                                      rr.at[b], sems[b]).wait()

                @pl.when(ci + 1 < NCH)
                def _():
                    fetch(ci + 1, 1 - b)

                base = base0 + ci * C
                for v in range(C // 16):
                    d8v[pl.ds(v * 16, 16)] = lax.shift_right_logical(
                        dstA[pl.ds(off + v * 16, 16)], 3)

                def grp(g):
                    logits = [jnp.zeros((16,), f32) for _ in range(heads)]
                    ars = [jnp.zeros((16,), f32) for _ in range(heads)]
                    for j in range(16):
                        e = g * 16 + j
                        uv = rr[b, e, pl.ds(D, 16)]
                        for hh in range(heads):
                            acc = None
                            for v in range(HV):
                                o2 = (hh * HV + v) * 16
                                s = (rl[b, e, pl.ds(o2, 16)]
                                     + rr[b, e, pl.ds(o2, 16)])
                                t = jnp.where(s >= 0.0, s, 0.2 * s)
                                pt = t * attr[hh * HV + v]
                                acc = pt if acc is None else acc + pt
                            lgv = _hsum16(acc)
                            logits[hh] = jnp.where(iot == j, lgv, logits[hh])
                            ars[hh] = jnp.where(
                                iot == j, _lane_shuffle(uv, _splat_i(hh)),
                                ars[hh])
                    eidv = iot + (base + g * 16)
                    wm = [jnp.where(eidv < E2,
                                    jnp.exp(logits[hh] - ars[hh] - ghs[hh]),
                                    0.0)
                          for hh in range(heads)]
                    for hh in range(heads):
                        wbuf[hh, pl.ds(g * 16, 16)] = wm[hh]
                    d8f = jnp.bitwise_and(
                        dstA[pl.ds(off + g * 16, 16)], 7).astype(f32)
                    for j in range(16):
                        w4 = jnp.zeros((16,), f32)
                        for hh in range(heads):
                            whj = _lane_shuffle(wm[hh], _splat_i(j))
                            w4 = jnp.where(iot == hh, whj, w4)
                        dself = _lane_shuffle(d8f, _splat_i(j))
                        for v in range(8):
                            ind = jnp.maximum(
                                1.0 - jnp.abs(dself - float(v)), 0.0)
                            val_d[g * 16 + j, pl.ds(v * 16, 16)] = w4 * ind

                plsc.parallel_loop(0, C // 16)(grp)
                pltpu.sync_copy(wbuf, w_hbm.at[wid * NCH + ci])
                pltpu.sync_copy(val_d, acc_d.at[d8v], add=True)
            return 0

        lax.fori_loop(0, NCH // 2, pair, 0)
        plsc.subcore_barrier()

        @pl.when(cid == 0)
        def _():
            pltpu.sync_copy(acc_d.at[pl.ds(sid * ZR8, ZR8)],
                            dlo_hbm.at[pl.ds(sid * ZR8, ZR8)])

        @pl.when(cid == 1)
        def _():
            pltpu.sync_copy(acc_d.at[pl.ds(sid * ZR8, ZR8)],
                            dhi_hbm.at[pl.ds(sid * ZR8, ZR8)])

    fn = pl.kernel(
        body,
        out_type=(jax.ShapeDtypeStruct((EP // C, heads, C), f32),
                  jax.ShapeDtypeStruct((ND8, 128), f32),
                  jax.ShapeDtypeStruct((ND8, 128), f32)),
        mesh=plsc.VectorSubcoreMesh(**_MESH),
        scratch_types=[
            pltpu.VMEM((PW,), i32),
            pltpu.VMEM((PW,), i32),
            pltpu.VMEM((C,), i32),
            pltpu.VMEM((2, C, D), f32),
            pltpu.VMEM((2, C, DA), f32),
            pltpu.VMEM((D,), f32),
            pltpu.VMEM((16,), f32),
            pltpu.VMEM((heads, C), f32),
            pltpu.VMEM((C, 128), f32),
            pltpu.VMEM_SHARED((ND8, 128), f32),
            pltpu.SemaphoreType.DMA,
            pltpu.SemaphoreType.DMA,
        ],
    )
    return fn(xl_t, xra_t, g16, srcp, dstp, attf, jnp.zeros((ZR8, 128), f32))


def _sc_pass2(xlo_t, xhi_t, w_flat, srcp, dstp, heads, Dh, zeros_blk,
              split_edges=False):
    ACC_W = Dh
    PW = EP // (32 if split_edges else 16)
    NCH = PW // C
    ZR = NP // 16
    wlo = (0, 1) if heads == 4 else (0,)
    whi = (2, 3) if heads == 4 else (0,)
    NWR = len(wlo)

    def body(xlo_hbm, xhi_hbm, w_hbm, src_hbm, dst_hbm, z_hbm,
             out_lo, out_hi, srcA, dstA, dstv, rows, wv, acc, semA, semB):
        cid = lax.axis_index("c")
        sid = lax.axis_index("s")
        if split_edges:
            base0 = (sid * 2 + cid) * PW
            cg0 = (sid * 2 + cid) * NCH
        else:
            base0 = sid * PW
            cg0 = sid * NCH
        pltpu.sync_copy(src_hbm.at[pl.ds(base0, PW)], srcA)
        pltpu.sync_copy(dst_hbm.at[pl.ds(base0, PW)], dstA)
        pltpu.sync_copy(z_hbm, acc.at[pl.ds(sid * ZR, ZR)])
        plsc.subcore_barrier()
        sems = [semA, semB]

        def run_half(tab_hbm, k0, out_hbm):
            def fetch(ci, b):
                off = ci * C
                pltpu.async_copy(tab_hbm.at[srcA.at[pl.ds(off, C)]],
                                 rows.at[b], sems[b])
                pltpu.async_copy(w_hbm.at[cg0 + ci], wv.at[b], sems[b])

            fetch(0, 0)

            def pair(c2, _):
                for b in range(2):
                    ci = c2 * 2 + b
                    off = ci * C
                    pltpu.make_async_copy(tab_hbm.at[srcA.at[pl.ds(off, C)]],
                                          rows.at[b], sems[b]).wait()
                    pltpu.make_async_copy(w_hbm.at[cg0 + ci], wv.at[b],
                                          sems[b]).wait()

                    @pl.when(ci + 1 < NCH)
                    def _():
                        fetch(ci + 1, 1 - b)

                    for v in range(C // 16):
                        dstv[pl.ds(v * 16, 16)] = dstA[pl.ds(off + v * 16, 16)]

                    def grp(g):
                        wrow0 = wv[b, k0, pl.ds(g * 16, 16)]
                        wrow1 = wv[b, k0 + NWR - 1, pl.ds(g * 16, 16)]
                        for j in range(16):
                            e = g * 16 + j
                            w0 = _lane_shuffle(wrow0, _splat_i(j))
                            w1 = _lane_shuffle(wrow1, _splat_i(j))
                            for v in range(Dh // 16):
                                wsel = w0 if v < (Dh // 32) else w1
                                rows[b, e, pl.ds(v * 16, 16)] = (
                                    rows[b, e, pl.ds(v * 16, 16)] * wsel)

                    plsc.parallel_loop(0, C // 16)(grp)
                    pltpu.sync_copy(rows.at[b], acc.at[dstv], add=True)
                return 0

            lax.fori_loop(0, NCH // 2, pair, 0)
            plsc.subcore_barrier()
            pltpu.sync_copy(acc.at[pl.ds(sid * ZR, ZR)],
                            out_hbm.at[pl.ds(sid * ZR, ZR)])

        @pl.when(cid == 0)
        def _():
            run_half(xlo_hbm, 0, out_lo)

        @pl.when(cid == 1)
        def _():
            run_half(xhi_hbm, 2 if heads == 4 else 0, out_hi)

    fn = pl.kernel(
        body,
        out_type=(jax.ShapeDtypeStruct((NP, ACC_W), f32),
                  jax.ShapeDtypeStruct((NP, ACC_W), f32)),
        mesh=plsc.VectorSubcoreMesh(**_MESH),
        scratch_types=[
            pltpu.VMEM((PW,), i32),
            pltpu.VMEM((PW,), i32),
            pltpu.VMEM((C,), i32),
            pltpu.VMEM((2, C, Dh), f32),
            pltpu.VMEM((2, heads, C), f32),
            pltpu.VMEM_SHARED((NP, ACC_W), f32),
            pltpu.SemaphoreType.DMA,
            pltpu.SemaphoreType.DMA,
        ],
    )
    return fn(xlo_t, xhi_t, w_flat, srcp, dstp, zeros_blk)


def _sc_decode(u_t, ud_t, s_idx, d_idx):
    PW = P // 32
    NCH = PW // C

    def body(u_hbm, ud_hbm, s_hbm, d_hbm, out_hbm, sA, dA, us, udv, ov,
             semA, semB):
        cid = lax.axis_index("c")
        sid = lax.axis_index("s")
        wid = sid * 2 + cid
        base0 = wid * PW
        iot = lax.iota(i32, 16)
        pltpu.sync_copy(s_hbm.at[pl.ds(base0, PW)], sA)
        pltpu.sync_copy(d_hbm.at[pl.ds(base0, PW)], dA)
        sems = [semA, semB]

        def fetch(ci, b):
            off = ci * C
            pltpu.async_copy(u_hbm.at[sA.at[pl.ds(off, C)]], us.at[b], sems[b])
            pltpu.async_copy(ud_hbm.at[dA.at[pl.ds(off, C)]], udv.at[b],
                             sems[b])

        fetch(0, 0)

        def pair(c2, _):
            for b in range(2):
                ci = c2 * 2 + b
                off = ci * C
                pltpu.make_async_copy(u_hbm.at[sA.at[pl.ds(off, C)]],
                                      us.at[b], sems[b]).wait()
                pltpu.make_async_copy(ud_hbm.at[dA.at[pl.ds(off, C)]],
                                      udv.at[b], sems[b]).wait()

                @pl.when(ci + 1 < NCH)
                def _():
                    fetch(ci + 1, 1 - b)

                def grp(g):
                    sreg = jnp.zeros((16,), f32)
                    for j in range(16):
                        e = g * 16 + j
                        ps = [us[b, e, pl.ds(v * 16, 16)]
                              * udv[b, e, pl.ds(v * 16, 16)]
                              for v in range(16)]
                        while len(ps) > 1:
                            ps = [ps[k] + ps[k + 1]
                                  for k in range(0, len(ps) - 1, 2)] + (
                                      [ps[-1]] if len(ps) % 2 else [])
                        sreg = jnp.where(iot == j, _hsum16(ps[0]), sreg)
                    ov[pl.ds(off + g * 16, 16)] = sreg

                plsc.parallel_loop(0, C // 16)(grp)
            return 0

        lax.fori_loop(0, NCH // 2, pair, 0)
        pltpu.sync_copy(ov, out_hbm.at[pl.ds(base0, PW)])

    fn = pl.kernel(
        body,
        out_type=jax.ShapeDtypeStruct((P,), f32),
        mesh=plsc.VectorSubcoreMesh(**_MESH),
        scratch_types=[
            pltpu.VMEM((PW,), i32),
            pltpu.VMEM((PW,), i32),
            pltpu.VMEM((2, C, 256), f32),
            pltpu.VMEM((2, C, 256), f32),
            pltpu.VMEM((PW,), f32),
            pltpu.SemaphoreType.DMA,
            pltpu.SemaphoreType.DMA,
        ],
    )
    return fn(u_t, ud_t, s_idx, d_idx)


def kernel(x, edge_index, edge_pairs, W0, b0, ln0_g, ln0_b, Wl1, bl1, Wr1, br1,
           att1, bias1, ln1_g, ln1_b, Wl2, bl2, Wr2, br2, att2, bias2, Wm1, bm1,
           lnm_g, lnm_b, Wm2, bm2, logit_alpha, temperature):
    r = lambda v: jnp.reshape(v, (1, -1))
    x_pad = jnp.pad(x.astype(f32), ((0, NP - N), (0, 0)))
    loops = jnp.arange(N, dtype=i32)
    srcp = jnp.concatenate([edge_index[0].astype(i32), loops,
                            jnp.zeros((EP - E2,), i32)])
    dstp = jnp.concatenate([edge_index[1].astype(i32), loops,
                            jnp.zeros((EP - E2,), i32)])
    s_idx = edge_pairs[:, 0].astype(i32)
    d_idx = edge_pairs[:, 1].astype(i32)

    attf1 = jnp.abs(att1).reshape(256)
    mask1 = (jnp.arange(256)[:, None] // 64) == jnp.arange(4)[None, :]
    M1 = jnp.where(mask1, attf1[:, None], 0.0)
    att1_flat = att1.reshape(256)
    att2_abs = jnp.abs(att2).reshape(128, 1)
    att2_flat = att2.reshape(128)

    a = jax.nn.sigmoid(logit_alpha)[0]
    cg = jnp.sqrt(a).reshape(1, 1)
    cf = jnp.sqrt(1.0 - a).reshape(1, 1)
    tmp = jnp.reshape(temperature.astype(f32), (1, 1))

    x_proj = pl.pallas_call(
        _t1_body, grid=(GR,),
        in_specs=[_rows(128), _full((128, 256)), _full((1, 256)),
                  _full((1, 256)), _full((1, 256))],
        out_specs=_rows(256),
        out_shape=jax.ShapeDtypeStruct((NP, 256), f32),
    )(x_pad, W0, r(b0), r(ln0_g), r(ln0_b))

    xl1, xr1a, zfc, g16 = pl.pallas_call(
        _t2_body, grid=(GR,),
        in_specs=[_rows(256), _full((256, 256)), _full((1, 256)),
                  _full((256, 256)), _full((1, 256)), _full((256, 256)),
                  _full((1, 256)), _full((1, 256)), _full((1, 256)),
                  _full((256, 128)), _full((1, 128)), _full((256, 4)),
                  _full((1, 1))],
        out_specs=[_rows(256), _rows(384), _rows(128),
                   pl.BlockSpec((1, 16), lambda i: (0, 0))],
        out_shape=[jax.ShapeDtypeStruct((NP, 256), f32),
                   jax.ShapeDtypeStruct((NP, 384), f32),
                   jax.ShapeDtypeStruct((NP, 128), f32),
                   jax.ShapeDtypeStruct((1, 16), f32)],
    )(x_proj, Wl1, r(bl1), Wr1, r(br1), Wm1, r(bm1), r(lnm_g), r(lnm_b),
      Wm2, r(bm2), M1, cf)

    z128 = jnp.zeros((NP // 16, 128), f32)
    w1, d1lo, d1hi = _sc_pass1(xl1, xr1a, g16.reshape(16), srcp, dstp,
                               att1_flat, heads=4, D=256, DA=384)
    acc_lo, acc_hi = _sc_pass2(xl1[:, :128], xl1[:, 128:], w1, srcp, dstp,
                               heads=4, Dh=128, zeros_blk=z128)

    xl2, xr2a, g2_16 = pl.pallas_call(
        _t3_body, grid=(GR,),
        in_specs=[_rows(128), _rows(128), _rows(16), _rows(16),
                  _full((1, 256)), _full((1, 256)),
                  _full((1, 256)), _full((256, 128)), _full((1, 128)),
                  _full((256, 128)), _full((1, 128)), _full((128, 1))],
        out_specs=[_rows(128), _rows(256),
                   pl.BlockSpec((1, 16), lambda i: (0, 0))],
        out_shape=[jax.ShapeDtypeStruct((NP, 128), f32),
                   jax.ShapeDtypeStruct((NP, 256), f32),
                   jax.ShapeDtypeStruct((1, 16), f32)],
    )(acc_lo, acc_hi, d1lo.reshape(NP, 16), d1hi.reshape(NP, 16), r(bias1),
      r(ln1_g), r(ln1_b), Wl2, r(bl2), Wr2, r(br2), att2_abs)

    w2, d2lo, d2hi = _sc_pass1(xl2, xr2a, g2_16.reshape(16), srcp, dstp,
                               att2_flat, heads=1, D=128, DA=256)
    acc2_lo, acc2_hi = _sc_pass2(xl2, xl2, w2, srcp, dstp,
                                 heads=1, Dh=128, zeros_blk=z128,
                                 split_edges=True)

    u, ud = pl.pallas_call(
        _t4_body, grid=(GR,),
        in_specs=[_rows(128), _rows(128), _rows(16), _rows(16),
                  _full((1, 128)), _rows(128),
                  _full((1, 1)), _full((1, 1))],
        out_specs=[_rows(256), _rows(256)],
        out_shape=[jax.ShapeDtypeStruct((NP, 256), f32),
                   jax.ShapeDtypeStruct((NP, 256), f32)],
    )(acc2_lo, acc2_hi, d2lo.reshape(NP, 16), d2hi.reshape(NP, 16), r(bias2),
      zfc, cg, tmp)

    return _sc_decode(u, ud, s_idx, d_idx)


# speedup vs baseline: 1.4826x; 1.4826x over previous
import functools

import jax
import jax.numpy as jnp
from jax import lax
from jax.experimental import pallas as pl
from jax.experimental.pallas import tpu as pltpu
from jax.experimental.pallas import tpu_sc as plsc

f32 = jnp.float32
i32 = jnp.int32

N = 10000
NP = 10240
E2 = 170000
EP = 172032
P = 262144
C = 64
BR = 512
GR = NP // BR

_MESH = dict(core_axis_name="c", subcore_axis_name="s")


def _ln(y, g, b):
    mu = jnp.mean(y, axis=-1, keepdims=True)
    var = jnp.mean((y - mu) ** 2, axis=-1, keepdims=True)
    return (y - mu) / jnp.sqrt(var + 1e-5) * g + b


def _t1_body(x_ref, w_ref, b_ref, g_ref, be_ref, o_ref):
    xb = x_ref[...]
    xb = jnp.where(jnp.isfinite(xb), xb, 0.0)
    y = jnp.dot(xb, w_ref[...], preferred_element_type=f32) + b_ref[...]
    o_ref[...] = jnp.maximum(_ln(y, g_ref[...], be_ref[...]), 0.0)


def _t2_body(xp_ref, wl_ref, bl_ref, wr_ref, br_ref, wm1_ref, bm1_ref,
             lg_ref, lb_ref, wm2_ref, bm2_ref, m1_ref, cf_ref,
             xl_ref, xra_ref, zfc_ref, g16_ref):
    xp = xp_ref[...]
    xl = jnp.dot(xp, wl_ref[...], preferred_element_type=f32) + bl_ref[...]
    xr = jnp.dot(xp, wr_ref[...], preferred_element_type=f32) + br_ref[...]
    xl_ref[...] = xl
    hm = jnp.maximum(_ln(jnp.dot(xp, wm1_ref[...], preferred_element_type=f32)
                         + bm1_ref[...], lg_ref[...], lb_ref[...]), 0.0)
    zf = jnp.dot(hm, wm2_ref[...], preferred_element_type=f32) + bm2_ref[...]
    nf = jnp.maximum(jnp.sqrt(jnp.sum(zf * zf, axis=-1, keepdims=True)), 1e-8)
    zfc_ref[...] = zf / nf * cf_ref[0, 0]
    m1 = m1_ref[...]
    ar4 = jnp.dot(jnp.abs(xr), m1, preferred_element_type=f32)
    xra_ref[...] = jnp.concatenate(
        [xr, jnp.pad(ar4, ((0, 0), (0, 124)))], axis=1)
    al4 = jnp.dot(jnp.abs(xl), m1, preferred_element_type=f32)
    gb = jnp.pad(jnp.max(al4, axis=0, keepdims=True), ((0, 0), (0, 12)))

    @pl.when(pl.program_id(0) == 0)
    def _():
        g16_ref[...] = gb

    @pl.when(pl.program_id(0) > 0)
    def _():
        g16_ref[...] = jnp.maximum(g16_ref[...], gb)


def _t3_body(lo_ref, hi_ref, dl_ref, dh_ref, b1_ref, lg_ref, lb_ref,
             wl2_ref, bl2_ref, wr2_ref, br2_ref, a2_ref,
             xl2_ref, xra_ref, g16_ref):
    num = jnp.concatenate([lo_ref[...], hi_ref[...]], axis=1)
    dinv = 1.0 / (dl_ref[...][:, 0:4] + dh_ref[...][:, 0:4] + 1e-30)
    scale = jnp.concatenate(
        [jnp.broadcast_to(dinv[:, h:h + 1], (BR, 64)) for h in range(4)], axis=1)
    h1 = jnp.maximum(_ln(num * scale + b1_ref[...], lg_ref[...], lb_ref[...]), 0.0)
    xl2 = jnp.dot(h1, wl2_ref[...], preferred_element_type=f32) + bl2_ref[...]
    xr2 = jnp.dot(h1, wr2_ref[...], preferred_element_type=f32) + br2_ref[...]
    xl2_ref[...] = xl2
    a2 = a2_ref[...]
    ar1 = jnp.dot(jnp.abs(xr2), a2, preferred_element_type=f32)
    xra_ref[...] = jnp.concatenate(
        [xr2, jnp.pad(ar1, ((0, 0), (0, 127)))], axis=1)
    al1 = jnp.dot(jnp.abs(xl2), a2, preferred_element_type=f32)
    gb = jnp.pad(jnp.max(al1, axis=0, keepdims=True), ((0, 0), (0, 15)))

    @pl.when(pl.program_id(0) == 0)
    def _():
        g16_ref[...] = gb

    @pl.when(pl.program_id(0) > 0)
    def _():
        g16_ref[...] = jnp.maximum(g16_ref[...], gb)


def _t4_body(lo_ref, hi_ref, dl_ref, dh_ref, b2_ref, zfc_ref, cg_ref, t_ref,
             u_ref, ud_ref):
    num = lo_ref[...] + hi_ref[...]
    den = dl_ref[...][:, 0:1] + dh_ref[...][:, 0:1]
    zg = num / (den + 1e-30) + b2_ref[...]
    ng = jnp.maximum(jnp.sqrt(jnp.sum(zg * zg, axis=-1, keepdims=True)), 1e-8)
    u = jnp.concatenate([zg / ng * cg_ref[0, 0], zfc_ref[...]], axis=1)
    u_ref[...] = u
    ud_ref[...] = u * t_ref[0, 0]


def _full(shape):
    return pl.BlockSpec(shape, lambda i: tuple(0 for _ in shape))


def _rows(w):
    return pl.BlockSpec((BR, w), lambda i: (i, 0))


def _splat_i(v):
    return jnp.full((16,), v, i32)


_GDN = lax.GatherDimensionNumbers(offset_dims=(), collapsed_slice_dims=(0,),
                                  start_index_map=(0,))


def _lane_shuffle(v, idx):
    return lax.gather(v, idx[:, None], _GDN, (1,),
                      mode=lax.GatherScatterMode.PROMISE_IN_BOUNDS)


def _hsum16(v):
    iot = lax.iota(i32, 16)
    for sh in (8, 4, 2, 1):
        v = v + _lane_shuffle(v, iot ^ sh)
    return v


def _sc_pass1(xl_t, xra_t, g16, srcp, dstp, attf, heads, D, DA):
    HV = D // heads // 16
    PW = EP // 32
    NCH = PW // C
    ND8 = NP // 8
    ZR8 = ND8 // 16

    def body(xl_hbm, xra_hbm, g_hbm, src_hbm, dst_hbm, att_hbm, z_hbm,
             w_hbm, dlo_hbm, dhi_hbm,
             srcA, dstA, d8v, rl, rr, attv, gv, wbuf, val_d, acc_d,
             semA, semB):
        cid = lax.axis_index("c")
        sid = lax.axis_index("s")
        wid = sid * 2 + cid
        base0 = wid * PW
        pltpu.sync_copy(att_hbm, attv)
        pltpu.sync_copy(g_hbm, gv)
        pltpu.sync_copy(src_hbm.at[pl.ds(base0, PW)], srcA)
        pltpu.sync_copy(dst_hbm.at[pl.ds(base0, PW)], dstA)
        pltpu.sync_copy(z_hbm, acc_d.at[pl.ds(sid * ZR8, ZR8)])
        iot = lax.iota(i32, 16)
        plsc.subcore_barrier()

        gva = gv[pl.ds(0, 16)]
        ghs = [_lane_shuffle(gva, _splat_i(hh)) for hh in range(heads)]

        attr = [attv[pl.ds(v * 16, 16)] for v in range(D // 16)]
        sems = [semA, semB]

        def fetch(ci, b):
            off = ci * C
            pltpu.async_copy(xl_hbm.at[srcA.at[pl.ds(off, C)]], rl.at[b],
                             sems[b])
            pltpu.async_copy(xra_hbm.at[dstA.at[pl.ds(off, C)]], rr.at[b],
                             sems[b])

        fetch(0, 0)

        def pair(c2, _):
            for b in range(2):
                ci = c2 * 2 + b
                off = ci * C
                pltpu.make_async_copy(xl_hbm.at[srcA.at[pl.ds(off, C)]],
                                      rl.at[b], sems[b]).wait()
                pltpu.make_async_copy(xra_hbm.at[dstA.at[pl.ds(off, C)]],
                                      rr.at[b], sems[b]).wait()

                @pl.when(ci + 1 < NCH)
                def _():
                    fetch(ci + 1, 1 - b)

                base = base0 + ci * C
                for v in range(C // 16):
                    d8v[pl.ds(v * 16, 16)] = lax.shift_right_logical(
                        dstA[pl.ds(off + v * 16, 16)], 3)

                def grp(g):
                    logits = [jnp.zeros((16,), f32) for _ in range(heads)]
                    ars = [jnp.zeros((16,), f32) for _ in range(heads)]
                    for j in range(16):
                        e = g * 16 + j
                        uv = rr[b, e, pl.ds(D, 16)]
                        for hh in range(heads):
                            ps = []
                            for v in range(HV):
                                o2 = (hh * HV + v) * 16
                                s = (rl[b, e, pl.ds(o2, 16)]
                                     + rr[b, e, pl.ds(o2, 16)])
                                t = jnp.maximum(s, 0.2 * s)
                                ps.append(t * attr[hh * HV + v])
                            while len(ps) > 1:
                                ps = [ps[k] + ps[k + 1]
                                      for k in range(0, len(ps) - 1, 2)] + (
                                          [ps[-1]] if len(ps) % 2 else [])
                            lgv = _hsum16(ps[0])
                            logits[hh] = jnp.where(iot == j, lgv, logits[hh])
                            ars[hh] = jnp.where(
                                iot == j, _lane_shuffle(uv, _splat_i(hh)),
                                ars[hh])
                    eidv = iot + (base + g * 16)
                    wm = [jnp.where(eidv < E2,
                                    jnp.exp(logits[hh] - ars[hh] - ghs[hh]),
                                    0.0)
                          for hh in range(heads)]
                    for hh in range(heads):
                        wbuf[hh, pl.ds(g * 16, 16)] = wm[hh]
                    d8f = jnp.bitwise_and(
                        dstA[pl.ds(off + g * 16, 16)], 7).astype(f32)
                    for j in range(16):
                        w4 = jnp.zeros((16,), f32)
                        for hh in range(heads):
                            whj = _lane_shuffle(wm[hh], _splat_i(j))
                            w4 = jnp.where(iot == hh, whj, w4)
                        dself = _lane_shuffle(d8f, _splat_i(j))
                        for v in range(8):
                            ind = jnp.maximum(
                                1.0 - jnp.abs(dself - float(v)), 0.0)
                            val_d[g * 16 + j, pl.ds(v * 16, 16)] = w4 * ind

                plsc.parallel_loop(0, C // 16)(grp)
                pltpu.sync_copy(wbuf, w_hbm.at[wid * NCH + ci])
                pltpu.sync_copy(val_d, acc_d.at[d8v], add=True)
            return 0

        lax.fori_loop(0, NCH // 2, pair, 0)
        plsc.subcore_barrier()

        @pl.when(cid == 0)
        def _():
            pltpu.sync_copy(acc_d.at[pl.ds(sid * ZR8, ZR8)],
                            dlo_hbm.at[pl.ds(sid * ZR8, ZR8)])

        @pl.when(cid == 1)
        def _():
            pltpu.sync_copy(acc_d.at[pl.ds(sid * ZR8, ZR8)],
                            dhi_hbm.at[pl.ds(sid * ZR8, ZR8)])

    fn = pl.kernel(
        body,
        out_type=(jax.ShapeDtypeStruct((EP // C, heads, C), f32),
                  jax.ShapeDtypeStruct((ND8, 128), f32),
                  jax.ShapeDtypeStruct((ND8, 128), f32)),
        mesh=plsc.VectorSubcoreMesh(**_MESH),
        scratch_types=[
            pltpu.VMEM((PW,), i32),
            pltpu.VMEM((PW,), i32),
            pltpu.VMEM((C,), i32),
            pltpu.VMEM((2, C, D), f32),
            pltpu.VMEM((2, C, DA), f32),
            pltpu.VMEM((D,), f32),
            pltpu.VMEM((16,), f32),
            pltpu.VMEM((heads, C), f32),
            pltpu.VMEM((C, 128), f32),
            pltpu.VMEM_SHARED((ND8, 128), f32),
            pltpu.SemaphoreType.DMA,
            pltpu.SemaphoreType.DMA,
        ],
    )
    return fn(xl_t, xra_t, g16, srcp, dstp, attf, jnp.zeros((ZR8, 128), f32))


def _sc_pass2(xlo_t, xhi_t, w_flat, srcp, dstp, heads, Dh, zeros_blk,
              split_edges=False):
    ACC_W = Dh
    PW = EP // (32 if split_edges else 16)
    NCH = PW // C
    ZR = NP // 16
    wlo = (0, 1) if heads == 4 else (0,)
    whi = (2, 3) if heads == 4 else (0,)
    NWR = len(wlo)

    def body(xlo_hbm, xhi_hbm, w_hbm, src_hbm, dst_hbm, z_hbm,
             out_lo, out_hi, srcA, dstA, dstv, rows, wv, acc, semA, semB):
        cid = lax.axis_index("c")
        sid = lax.axis_index("s")
        if split_edges:
            base0 = (sid * 2 + cid) * PW
            cg0 = (sid * 2 + cid) * NCH
        else:
            base0 = sid * PW
            cg0 = sid * NCH
        pltpu.sync_copy(src_hbm.at[pl.ds(base0, PW)], srcA)
        pltpu.sync_copy(dst_hbm.at[pl.ds(base0, PW)], dstA)
        pltpu.sync_copy(z_hbm, acc.at[pl.ds(sid * ZR, ZR)])
        plsc.subcore_barrier()
        sems = [semA, semB]

        def run_half(tab_hbm, k0, out_hbm):
            def fetch(ci, b):
                off = ci * C
                pltpu.async_copy(tab_hbm.at[srcA.at[pl.ds(off, C)]],
                                 rows.at[b], sems[b])
                pltpu.async_copy(w_hbm.at[cg0 + ci], wv.at[b], sems[b])

            fetch(0, 0)

            def pair(c2, _):
                for b in range(2):
                    ci = c2 * 2 + b
                    off = ci * C
                    pltpu.make_async_copy(tab_hbm.at[srcA.at[pl.ds(off, C)]],
                                          rows.at[b], sems[b]).wait()
                    pltpu.make_async_copy(w_hbm.at[cg0 + ci], wv.at[b],
                                          sems[b]).wait()

                    @pl.when(ci + 1 < NCH)
                    def _():
                        fetch(ci + 1, 1 - b)

                    for v in range(C // 16):
                        dstv[pl.ds(v * 16, 16)] = dstA[pl.ds(off + v * 16, 16)]

                    def grp(g):
                        wrow0 = wv[b, k0, pl.ds(g * 16, 16)]
                        wrow1 = wv[b, k0 + NWR - 1, pl.ds(g * 16, 16)]
                        for j in range(16):
                            e = g * 16 + j
                            w0 = _lane_shuffle(wrow0, _splat_i(j))
                            w1 = _lane_shuffle(wrow1, _splat_i(j))
                            for v in range(Dh // 16):
                                wsel = w0 if v < (Dh // 32) else w1
                                rows[b, e, pl.ds(v * 16, 16)] = (
                                    rows[b, e, pl.ds(v * 16, 16)] * wsel)

                    plsc.parallel_loop(0, C // 16)(grp)
                    pltpu.sync_copy(rows.at[b], acc.at[dstv], add=True)
                return 0

            lax.fori_loop(0, NCH // 2, pair, 0)
            plsc.subcore_barrier()
            pltpu.sync_copy(acc.at[pl.ds(sid * ZR, ZR)],
                            out_hbm.at[pl.ds(sid * ZR, ZR)])

        @pl.when(cid == 0)
        def _():
            run_half(xlo_hbm, 0, out_lo)

        @pl.when(cid == 1)
        def _():
            run_half(xhi_hbm, 2 if heads == 4 else 0, out_hi)

    fn = pl.kernel(
        body,
        out_type=(jax.ShapeDtypeStruct((NP, ACC_W), f32),
                  jax.ShapeDtypeStruct((NP, ACC_W), f32)),
        mesh=plsc.VectorSubcoreMesh(**_MESH),
        scratch_types=[
            pltpu.VMEM((PW,), i32),
            pltpu.VMEM((PW,), i32),
            pltpu.VMEM((C,), i32),
            pltpu.VMEM((2, C, Dh), f32),
            pltpu.VMEM((2, heads, C), f32),
            pltpu.VMEM_SHARED((NP, ACC_W), f32),
            pltpu.SemaphoreType.DMA,
            pltpu.SemaphoreType.DMA,
        ],
    )
    return fn(xlo_t, xhi_t, w_flat, srcp, dstp, zeros_blk)


def _sc_decode(u_t, ud_t, s_idx, d_idx):
    PW = P // 32
    NCH = PW // C

    def body(u_hbm, ud_hbm, s_hbm, d_hbm, out_hbm, sA, dA, us, udv, ov,
             semA, semB):
        cid = lax.axis_index("c")
        sid = lax.axis_index("s")
        wid = sid * 2 + cid
        base0 = wid * PW
        iot = lax.iota(i32, 16)
        pltpu.sync_copy(s_hbm.at[pl.ds(base0, PW)], sA)
        pltpu.sync_copy(d_hbm.at[pl.ds(base0, PW)], dA)
        sems = [semA, semB]

        def fetch(ci, b):
            off = ci * C
            pltpu.async_copy(u_hbm.at[sA.at[pl.ds(off, C)]], us.at[b], sems[b])
            pltpu.async_copy(ud_hbm.at[dA.at[pl.ds(off, C)]], udv.at[b],
                             sems[b])

        fetch(0, 0)

        def pair(c2, _):
            for b in range(2):
                ci = c2 * 2 + b
                off = ci * C
                pltpu.make_async_copy(u_hbm.at[sA.at[pl.ds(off, C)]],
                                      us.at[b], sems[b]).wait()
                pltpu.make_async_copy(ud_hbm.at[dA.at[pl.ds(off, C)]],
                                      udv.at[b], sems[b]).wait()

                @pl.when(ci + 1 < NCH)
                def _():
                    fetch(ci + 1, 1 - b)

                def grp(g):
                    sreg = jnp.zeros((16,), f32)
                    for j in range(16):
                        e = g * 16 + j
                        acc = us[b, e, pl.ds(0, 16)] * udv[b, e, pl.ds(0, 16)]
                        for v in range(1, 16):
                            acc = acc + (us[b, e, pl.ds(v * 16, 16)]
                                         * udv[b, e, pl.ds(v * 16, 16)])
                        sreg = jnp.where(iot == j, _hsum16(acc), sreg)
                    ov[pl.ds(off + g * 16, 16)] = sreg

                plsc.parallel_loop(0, C // 16)(grp)
            return 0

        lax.fori_loop(0, NCH // 2, pair, 0)
        pltpu.sync_copy(ov, out_hbm.at[pl.ds(base0, PW)])

    fn = pl.kernel(
        body,
        out_type=jax.ShapeDtypeStruct((P,), f32),
        mesh=plsc.VectorSubcoreMesh(**_MESH),
        scratch_types=[
            pltpu.VMEM((PW,), i32),
            pltpu.VMEM((PW,), i32),
            pltpu.VMEM((2, C, 256), f32),
            pltpu.VMEM((2, C, 256), f32),
            pltpu.VMEM((PW,), f32),
            pltpu.SemaphoreType.DMA,
            pltpu.SemaphoreType.DMA,
        ],
    )
    return fn(u_t, ud_t, s_idx, d_idx)


def kernel(x, edge_index, edge_pairs, W0, b0, ln0_g, ln0_b, Wl1, bl1, Wr1, br1,
           att1, bias1, ln1_g, ln1_b, Wl2, bl2, Wr2, br2, att2, bias2, Wm1, bm1,
           lnm_g, lnm_b, Wm2, bm2, logit_alpha, temperature):
    r = lambda v: jnp.reshape(v, (1, -1))
    x_pad = jnp.pad(x.astype(f32), ((0, NP - N), (0, 0)))
    loops = jnp.arange(N, dtype=i32)
    srcp = jnp.concatenate([edge_index[0].astype(i32), loops,
                            jnp.zeros((EP - E2,), i32)])
    dstp = jnp.concatenate([edge_index[1].astype(i32), loops,
                            jnp.zeros((EP - E2,), i32)])
    s_idx = edge_pairs[:, 0].astype(i32)
    d_idx = edge_pairs[:, 1].astype(i32)

    attf1 = jnp.abs(att1).reshape(256)
    mask1 = (jnp.arange(256)[:, None] // 64) == jnp.arange(4)[None, :]
    M1 = jnp.where(mask1, attf1[:, None], 0.0)
    att1_flat = att1.reshape(256)
    att2_abs = jnp.abs(att2).reshape(128, 1)
    att2_flat = att2.reshape(128)

    a = jax.nn.sigmoid(logit_alpha)[0]
    cg = jnp.sqrt(a).reshape(1, 1)
    cf = jnp.sqrt(1.0 - a).reshape(1, 1)
    tmp = jnp.reshape(temperature.astype(f32), (1, 1))

    x_proj = pl.pallas_call(
        _t1_body, grid=(GR,),
        in_specs=[_rows(128), _full((128, 256)), _full((1, 256)),
                  _full((1, 256)), _full((1, 256))],
        out_specs=_rows(256),
        out_shape=jax.ShapeDtypeStruct((NP, 256), f32),
    )(x_pad, W0, r(b0), r(ln0_g), r(ln0_b))

    xl1, xr1a, zfc, g16 = pl.pallas_call(
        _t2_body, grid=(GR,),
        in_specs=[_rows(256), _full((256, 256)), _full((1, 256)),
                  _full((256, 256)), _full((1, 256)), _full((256, 256)),
                  _full((1, 256)), _full((1, 256)), _full((1, 256)),
                  _full((256, 128)), _full((1, 128)), _full((256, 4)),
                  _full((1, 1))],
        out_specs=[_rows(256), _rows(384), _rows(128),
                   pl.BlockSpec((1, 16), lambda i: (0, 0))],
        out_shape=[jax.ShapeDtypeStruct((NP, 256), f32),
                   jax.ShapeDtypeStruct((NP, 384), f32),
                   jax.ShapeDtypeStruct((NP, 128), f32),
                   jax.ShapeDtypeStruct((1, 16), f32)],
    )(x_proj, Wl1, r(bl1), Wr1, r(br1), Wm1, r(bm1), r(lnm_g), r(lnm_b),
      Wm2, r(bm2), M1, cf)

    z128 = jnp.zeros((NP // 16, 128), f32)
    w1, d1lo, d1hi = _sc_pass1(xl1, xr1a, g16.reshape(16), srcp, dstp,
                               att1_flat, heads=4, D=256, DA=384)
    acc_lo, acc_hi = _sc_pass2(xl1[:, :128], xl1[:, 128:], w1, srcp, dstp,
                               heads=4, Dh=128, zeros_blk=z128)

    xl2, xr2a, g2_16 = pl.pallas_call(
        _t3_body, grid=(GR,),
        in_specs=[_rows(128), _rows(128), _rows(16), _rows(16),
                  _full((1, 256)), _full((1, 256)),
                  _full((1, 256)), _full((256, 128)), _full((1, 128)),
                  _full((256, 128)), _full((1, 128)), _full((128, 1))],
        out_specs=[_rows(128), _rows(256),
                   pl.BlockSpec((1, 16), lambda i: (0, 0))],
        out_shape=[jax.ShapeDtypeStruct((NP, 128), f32),
                   jax.ShapeDtypeStruct((NP, 256), f32),
                   jax.ShapeDtypeStruct((1, 16), f32)],
    )(acc_lo, acc_hi, d1lo.reshape(NP, 16), d1hi.reshape(NP, 16), r(bias1),
      r(ln1_g), r(ln1_b), Wl2, r(bl2), Wr2, r(br2), att2_abs)

    w2, d2lo, d2hi = _sc_pass1(xl2, xr2a, g2_16.reshape(16), srcp, dstp,
                               att2_flat, heads=1, D=128, DA=256)
    acc2_lo, acc2_hi = _sc_pass2(xl2, xl2, w2, srcp, dstp,
                                 heads=1, Dh=128, zeros_blk=z128,
                                 split_edges=True)

    u, ud = pl.pallas_call(
        _t4_body, grid=(GR,),
        in_specs=[_rows(128), _rows(128), _rows(16), _rows(16),
                  _full((1, 128)), _rows(128),
                  _full((1, 1)), _full((1, 1))],
        out_specs=[_rows(256), _rows(256)],
        out_shape=[jax.ShapeDtypeStruct((NP, 256), f32),
                   jax.ShapeDtypeStruct((NP, 256), f32)],
    )(acc2_lo, acc2_hi, d2lo.reshape(NP, 16), d2hi.reshape(NP, 16), r(bias2),
      zfc, cg, tmp)

    return _sc_decode(u, ud, s_idx, d_idx)
